# Initial kernel scaffold; baseline (speedup 1.0000x reference)
import jax
import jax.numpy as jnp
from jax.experimental import pallas as pl


def kernel(user_emb_w, item_emb_w, adj_values, users, pos_items, neg_items, adj_indices):
    raise NotImplementedError("write your pallas kernel here")


# speedup vs baseline: 1.4566x; 1.4566x over previous
import jax
import jax.numpy as jnp
from jax import lax
from jax.experimental import pallas as pl
from jax.experimental.pallas import tpu as pltpu
from jax.experimental.pallas import tpu_sc as plsc

N_USERS = 25000
N_ITEMS = 25000
N = N_USERS + N_ITEMS
HALF = 32
E = 800000
BATCH = 4096
DECAY = 1e-4

NC = 2
NS = 16
NP = 50048
NNP = 2 * NP
ROWS_TILE = NP // NS
EP_TILE = E // NS
CHUNK = 400
SUB = 80
NSUB = CHUNK // SUB
NCHUNKS = EP_TILE // CHUNK
BP_TILE = BATCH // NS


def _sc_body(x0, src1, dst1, val1, un, pn, nn, zrows,
             x1, x2, x3, diff_o, reg_o,
             acc, src_f, dst_f, dst_v, val_f, rows_v,
             idx0_f, idxb_f, nd_f, g_v, sco_f, reg_v, sem):
    c = lax.axis_index("c")
    s = lax.axis_index("s")
    iota = lax.iota(jnp.int32, 16)

    def spmm(in_ref, out_ref, interleaved):
        pltpu.sync_copy(zrows, acc.at[pl.ds(s * ROWS_TILE, ROWS_TILE)])
        plsc.subcore_barrier()

        def chunk(j, _):
            e0 = s * EP_TILE + j * CHUNK
            pltpu.sync_copy(src1.at[pl.ds(e0, CHUNK)], src_f)
            pltpu.sync_copy(dst1.at[pl.ds(e0, CHUNK)], dst_f)
            pltpu.sync_copy(val1.at[pl.ds(e0, CHUNK)], val_f)
            for q in range(CHUNK // 16):
                sl = pl.ds(q * 16, 16)
                v = src_f[sl]
                if interleaved:
                    src_f[sl] = v * 2 + c
                else:
                    src_f[sl] = v + c * NP
                dst_v[q // (SUB // 16), pl.ds((q % (SUB // 16)) * 16, 16)] = dst_f[sl]
            handles = [
                pltpu.async_copy(in_ref.at[src_f.at[pl.ds(k * SUB, SUB)]],
                                 rows_v.at[pl.ds(k * SUB, SUB)], sem)
                for k in range(NSUB)
            ]
            for h in handles:
                h.wait()

            def grp(g, _):
                eids = iota + g * 16
                vvec = plsc.load_gather(val_f, [eids])
                for d in range(HALF):
                    dv = jnp.full((16,), d, jnp.int32)
                    col = plsc.load_gather(rows_v, [eids, dv])
                    plsc.store_scatter(rows_v, [eids, dv], col * vvec)
                return 0

            lax.fori_loop(0, CHUNK // 16, grp, 0)

            for k in range(NSUB):
                pltpu.sync_copy(rows_v.at[pl.ds(k * SUB, SUB)],
                                acc.at[dst_v.at[k]], add=True)
            return 0

        lax.fori_loop(0, NCHUNKS, chunk, 0)
        plsc.subcore_barrier()
        pltpu.sync_copy(acc.at[pl.ds(s * ROWS_TILE, ROWS_TILE)],
                        out_ref.at[pl.ds(c * NP + s * ROWS_TILE, ROWS_TILE)])
        plsc.subcore_barrier()

    spmm(x0, x1, True)
    spmm(x1, x2, False)
    spmm(x2, x3, False)

    BP = 128
    reg = jnp.zeros((16,), jnp.float32)
    for p in range(2):
        for si, nodes_hbm in enumerate((un, pn, nn)):
            base = si * BP
            pltpu.sync_copy(nodes_hbm.at[pl.ds(s * BP_TILE + p * BP, BP)], nd_f)
            for q in range(BP // 16):
                sl = pl.ds(q * 16, 16)
                v = nd_f[sl]
                idx0_f[sl] = v * 2 + c
                idxb_f[sl] = v + c * NP
            pltpu.async_copy(x0.at[idx0_f], rows_v.at[pl.ds(base, BP)],
                             sem).wait()

            def regbody(i, r, base=base):
                a = rows_v[base + i, pl.ds(0, 16)]
                b = rows_v[base + i, pl.ds(16, 16)]
                return r + a * a + b * b

            reg = lax.fori_loop(0, BP, regbody, reg)

            for tab in (x1, x2, x3):
                pltpu.async_copy(tab.at[idxb_f], g_v, sem).wait()

                def addbody(i, _, base=base):
                    rows_v[base + i, pl.ds(0, 16)] = (
                        rows_v[base + i, pl.ds(0, 16)] + g_v[i, pl.ds(0, 16)])
                    rows_v[base + i, pl.ds(16, 16)] = (
                        rows_v[base + i, pl.ds(16, 16)] + g_v[i, pl.ds(16, 16)])
                    return 0

                lax.fori_loop(0, BP, addbody, 0)

        def score(g, _, p=p):
            eids = iota + g * 16
            ps = jnp.zeros((16,), jnp.float32)
            ns = jnp.zeros((16,), jnp.float32)
            for d in range(HALF):
                dv = jnp.full((16,), d, jnp.int32)
                cu = plsc.load_gather(rows_v, [eids, dv])
                cp = plsc.load_gather(rows_v, [eids + BP, dv])
                cn = plsc.load_gather(rows_v, [eids + 2 * BP, dv])
                ps = ps + cu * cp
                ns = ns + cu * cn
            plsc.store_scatter(sco_f, [eids + p * BP], (ps - ns) * 0.0625)
            return 0

        lax.fori_loop(0, BP // 16, score, 0)

    reg_v[...] = reg
    pltpu.sync_copy(reg_v, reg_o.at[pl.ds((c * NS + s) * 16, 16)])
    pltpu.sync_copy(sco_f, diff_o.at[pl.ds((c * NS + s) * BP_TILE, BP_TILE)])


def _tc_final(diff_ref, reg_ref, loss_ref, bpr_ref):
    sd = diff_ref[0, :] + diff_ref[1, :]
    bpr = -jnp.mean(jax.nn.log_sigmoid(sd))
    reg = jnp.sum(reg_ref[...]) / BATCH
    loss_ref[0, 0] = bpr + DECAY * reg
    bpr_ref[0, 0] = bpr


@jax.jit
def kernel(user_emb_w, item_emb_w, adj_values, users, pos_items, neg_items, adj_indices):
    x0 = jnp.concatenate([user_emb_w, item_emb_w], axis=0).reshape(2 * N, HALF)
    src = adj_indices[0].astype(jnp.int32)
    dst = adj_indices[1].astype(jnp.int32)
    un = users.astype(jnp.int32)
    pn = pos_items.astype(jnp.int32) + N_USERS
    nn = neg_items.astype(jnp.int32) + N_USERS
    zrows = jnp.zeros((ROWS_TILE, HALF), jnp.float32)

    mesh = plsc.VectorSubcoreMesh(core_axis_name="c", subcore_axis_name="s",
                                  num_cores=NC, num_subcores=NS)
    f32 = jnp.float32
    i32 = jnp.int32
    sc = pl.kernel(
        _sc_body,
        out_type=(
            jax.ShapeDtypeStruct((NNP, HALF), f32),
            jax.ShapeDtypeStruct((NNP, HALF), f32),
            jax.ShapeDtypeStruct((NNP, HALF), f32),
            jax.ShapeDtypeStruct((NC * BATCH,), f32),
            jax.ShapeDtypeStruct((NC * NS * 16,), f32),
        ),
        mesh=mesh,
        compiler_params=pltpu.CompilerParams(needs_layout_passes=False,
                                             use_tc_tiling_on_sc=False),
        scratch_types=[
            pltpu.VMEM_SHARED((NP, HALF), f32),
            pltpu.VMEM((CHUNK,), i32),
            pltpu.VMEM((CHUNK,), i32),
            pltpu.VMEM((NSUB, SUB), i32),
            pltpu.VMEM((CHUNK,), f32),
            pltpu.VMEM((CHUNK, HALF), f32),
            pltpu.VMEM((128,), i32),
            pltpu.VMEM((128,), i32),
            pltpu.VMEM((128,), i32),
            pltpu.VMEM((128, HALF), f32),
            pltpu.VMEM((BP_TILE,), f32),
            pltpu.VMEM((16,), f32),
            pltpu.SemaphoreType.DMA,
        ],
    )
    x1, x2, x3, diff_p, reg_p = sc(x0, src, dst, adj_values, un, pn, nn, zrows)

    diff = diff_p.reshape(NC, BATCH)
    regs = reg_p.reshape(4, 128)
    loss, bpr = pl.pallas_call(
        _tc_final,
        out_shape=(
            jax.ShapeDtypeStruct((1, 1), f32),
            jax.ShapeDtypeStruct((1, 1), f32),
        ),
        out_specs=(pl.BlockSpec(memory_space=pltpu.SMEM),
                   pl.BlockSpec(memory_space=pltpu.SMEM)),
    )(diff, regs)
    return (loss[0, 0], bpr[0, 0])


# speedup vs baseline: 1.6289x; 1.1183x over previous
import jax
import jax.numpy as jnp
from jax import lax
from jax.experimental import pallas as pl
from jax.experimental.pallas import tpu as pltpu
from jax.experimental.pallas import tpu_sc as plsc

N_USERS = 25000
N_ITEMS = 25000
N = N_USERS + N_ITEMS
HALF = 32
E = 800000
BATCH = 4096
DECAY = 1e-4

NC = 2
NS = 16
NP = 50048
NNP = 2 * NP
ROWS_TILE = NP // NS
EP_TILE = E // NS
CHUNK = 400
SUB = 80
NSUB = CHUNK // SUB
NCHUNKS = EP_TILE // CHUNK
BP_TILE = BATCH // NS


def _sc_body(x0, pk, un, pn, nn, zrows,
             x1, x2, x3, diff_o, reg_o,
             acc, stg0, stg1, dv0, dv1, rows0, rows1,
             idx0_f, idxb_f, nd_f, sco_f, reg_v,
             stage_sem, gath_sem, scat0_sem, scat1_sem):
    c = lax.axis_index("c")
    s = lax.axis_index("s")
    iota = lax.iota(jnp.int32, 16)
    stg = (stg0, stg1)
    dvb = (dv0, dv1)
    rows = (rows0, rows1)
    scat_sem = (scat0_sem, scat1_sem)
    LAST = NCHUNKS - 1

    def spmm(in_ref, out_ref, interleaved):
        pltpu.sync_copy(zrows, acc.at[pl.ds(s * ROWS_TILE, ROWS_TILE)])
        plsc.subcore_barrier()

        def fire_stage(jj, b):
            off = (s * NCHUNKS + jj) * (3 * CHUNK)
            pltpu.async_copy(pk.at[pl.ds(off, 3 * CHUNK)], stg[b], stage_sem)

        def wait_stage(b):
            pltpu.make_async_copy(pk.at[pl.ds(0, 3 * CHUNK)], stg[b],
                                  stage_sem).wait()

        def adjust(b):
            sb, db = stg[b], dvb[b]
            for q in range(CHUNK // 16):
                sl = pl.ds(q * 16, 16)
                v = sb[sl]
                if interleaved:
                    sb[sl] = v * 2 + c
                else:
                    sb[sl] = v + c * NP
                db[q // (SUB // 16), pl.ds((q % (SUB // 16)) * 16, 16)] = (
                    sb[pl.ds(CHUNK + q * 16, 16)])

        def fire_gath(b):
            for k in range(NSUB):
                pltpu.async_copy(in_ref.at[stg[b].at[pl.ds(k * SUB, SUB)]],
                                 rows[b].at[pl.ds(k * SUB, SUB)], gath_sem)

        def wait_gath(b):
            for k in range(NSUB):
                pltpu.make_async_copy(
                    in_ref.at[stg[b].at[pl.ds(k * SUB, SUB)]],
                    rows[b].at[pl.ds(k * SUB, SUB)], gath_sem).wait()

        def fire_scat(b):
            for k in range(NSUB):
                pltpu.async_copy(rows[b].at[pl.ds(k * SUB, SUB)],
                                 acc.at[dvb[b].at[k]], scat_sem[b], add=True)

        def wait_scat(b):
            for k in range(NSUB):
                pltpu.make_async_copy(rows[b].at[pl.ds(k * SUB, SUB)],
                                      acc.at[dvb[b].at[k]],
                                      scat_sem[b]).wait()

        def compute(b):
            sb, rb = stg[b], rows[b]

            def grp(g, _):
                eids = iota + g * 16
                vvec = plsc.bitcast(
                    plsc.load_gather(sb, [eids + 2 * CHUNK]), jnp.float32)
                for d in range(HALF):
                    dv = jnp.full((16,), d, jnp.int32)
                    col = plsc.load_gather(rb, [eids, dv])
                    plsc.store_scatter(rb, [eids, dv], col * vvec)
                return 0

            lax.fori_loop(0, CHUNK // 16, grp, 0)

        def body(jj, b, first, last):
            nb = 1 - b
            if not last:
                fire_stage(jj + 1, nb)
            wait_gath(b)
            compute(b)
            if not last:
                wait_stage(nb)
                if not first:
                    wait_scat(nb)
                adjust(nb)
                fire_gath(nb)
            fire_scat(b)

        fire_stage(0, 0)
        wait_stage(0)
        adjust(0)
        fire_gath(0)
        body(jnp.int32(0), 0, True, False)
        body(jnp.int32(1), 1, False, False)

        def looped(t, _):
            body(2 * t + 2, 0, False, False)
            body(2 * t + 3, 1, False, False)
            return 0

        lax.fori_loop(0, (NCHUNKS - 3) // 2, looped, 0)
        body(jnp.int32(LAST), 0, False, True)
        wait_scat(1)
        wait_scat(0)

        plsc.subcore_barrier()
        pltpu.sync_copy(acc.at[pl.ds(s * ROWS_TILE, ROWS_TILE)],
                        out_ref.at[pl.ds(c * NP + s * ROWS_TILE, ROWS_TILE)])
        plsc.subcore_barrier()

    spmm(x0, x1, True)
    spmm(x1, x2, False)
    spmm(x2, x3, False)

    BP = 128
    reg = jnp.zeros((16,), jnp.float32)
    for p in range(2):
        for si, nodes_hbm in enumerate((un, pn, nn)):
            base = si * BP
            pltpu.sync_copy(nodes_hbm.at[pl.ds(s * BP_TILE + p * BP, BP)], nd_f)
            for q in range(BP // 16):
                sl = pl.ds(q * 16, 16)
                v = nd_f[sl]
                idx0_f[sl] = v * 2 + c
                idxb_f[sl] = v + c * NP
            pltpu.async_copy(x0.at[idx0_f], rows0.at[pl.ds(base, BP)],
                             gath_sem).wait()

            def regbody(i, r, base=base):
                a = rows0[base + i, pl.ds(0, 16)]
                b = rows0[base + i, pl.ds(16, 16)]
                return r + a * a + b * b

            reg = lax.fori_loop(0, BP, regbody, reg)

            for tab in (x1, x2, x3):
                pltpu.async_copy(tab.at[idxb_f], rows1.at[pl.ds(0, BP)],
                                 gath_sem).wait()

                def addbody(i, _, base=base):
                    rows0[base + i, pl.ds(0, 16)] = (
                        rows0[base + i, pl.ds(0, 16)] + rows1[i, pl.ds(0, 16)])
                    rows0[base + i, pl.ds(16, 16)] = (
                        rows0[base + i, pl.ds(16, 16)] + rows1[i, pl.ds(16, 16)])
                    return 0

                lax.fori_loop(0, BP, addbody, 0)

        def score(g, _, p=p):
            eids = iota + g * 16
            ps = jnp.zeros((16,), jnp.float32)
            ns = jnp.zeros((16,), jnp.float32)
            for d in range(HALF):
                dv = jnp.full((16,), d, jnp.int32)
                cu = plsc.load_gather(rows0, [eids, dv])
                cp = plsc.load_gather(rows0, [eids + BP, dv])
                cn = plsc.load_gather(rows0, [eids + 2 * BP, dv])
                ps = ps + cu * cp
                ns = ns + cu * cn
            plsc.store_scatter(sco_f, [eids + p * BP], (ps - ns) * 0.0625)
            return 0

        lax.fori_loop(0, BP // 16, score, 0)

    reg_v[...] = reg
    pltpu.sync_copy(reg_v, reg_o.at[pl.ds((c * NS + s) * 16, 16)])
    pltpu.sync_copy(sco_f, diff_o.at[pl.ds((c * NS + s) * BP_TILE, BP_TILE)])


def _tc_final(diff_ref, reg_ref, loss_ref, bpr_ref):
    sd = diff_ref[0, :] + diff_ref[1, :]
    bpr = -jnp.mean(jax.nn.log_sigmoid(sd))
    reg = jnp.sum(reg_ref[...]) / BATCH
    loss_ref[0, 0] = bpr + DECAY * reg
    bpr_ref[0, 0] = bpr


@jax.jit
def kernel(user_emb_w, item_emb_w, adj_values, users, pos_items, neg_items, adj_indices):
    x0 = jnp.concatenate([user_emb_w, item_emb_w], axis=0).reshape(2 * N, HALF)
    src = adj_indices[0].astype(jnp.int32).reshape(NS, NCHUNKS, CHUNK)
    dst = adj_indices[1].astype(jnp.int32).reshape(NS, NCHUNKS, CHUNK)
    vali = jax.lax.bitcast_convert_type(adj_values, jnp.int32).reshape(
        NS, NCHUNKS, CHUNK)
    pk = jnp.stack((src, dst, vali), axis=2).reshape(-1)
    un = users.astype(jnp.int32)
    pn = pos_items.astype(jnp.int32) + N_USERS
    nn = neg_items.astype(jnp.int32) + N_USERS
    zrows = jnp.zeros((ROWS_TILE, HALF), jnp.float32)

    mesh = plsc.VectorSubcoreMesh(core_axis_name="c", subcore_axis_name="s",
                                  num_cores=NC, num_subcores=NS)
    f32 = jnp.float32
    i32 = jnp.int32
    sc = pl.kernel(
        _sc_body,
        out_type=(
            jax.ShapeDtypeStruct((NNP, HALF), f32),
            jax.ShapeDtypeStruct((NNP, HALF), f32),
            jax.ShapeDtypeStruct((NNP, HALF), f32),
            jax.ShapeDtypeStruct((NC * BATCH,), f32),
            jax.ShapeDtypeStruct((NC * NS * 16,), f32),
        ),
        mesh=mesh,
        compiler_params=pltpu.CompilerParams(needs_layout_passes=False,
                                             use_tc_tiling_on_sc=False),
        scratch_types=[
            pltpu.VMEM_SHARED((NP, HALF), f32),
            pltpu.VMEM((3 * CHUNK,), i32),
            pltpu.VMEM((3 * CHUNK,), i32),
            pltpu.VMEM((NSUB, SUB), i32),
            pltpu.VMEM((NSUB, SUB), i32),
            pltpu.VMEM((CHUNK, HALF), f32),
            pltpu.VMEM((CHUNK, HALF), f32),
            pltpu.VMEM((128,), i32),
            pltpu.VMEM((128,), i32),
            pltpu.VMEM((128,), i32),
            pltpu.VMEM((BP_TILE,), f32),
            pltpu.VMEM((16,), f32),
            pltpu.SemaphoreType.DMA,
            pltpu.SemaphoreType.DMA,
            pltpu.SemaphoreType.DMA,
            pltpu.SemaphoreType.DMA,
        ],
    )
    x1, x2, x3, diff_p, reg_p = sc(x0, pk, un, pn, nn, zrows)

    diff = diff_p.reshape(NC, BATCH)
    regs = reg_p.reshape(4, 128)
    loss, bpr = pl.pallas_call(
        _tc_final,
        out_shape=(
            jax.ShapeDtypeStruct((1, 1), f32),
            jax.ShapeDtypeStruct((1, 1), f32),
        ),
        out_specs=(pl.BlockSpec(memory_space=pltpu.SMEM),
                   pl.BlockSpec(memory_space=pltpu.SMEM)),
    )(diff, regs)
    return (loss[0, 0], bpr[0, 0])


# speedup vs baseline: 11.1452x; 6.8421x over previous
import jax
import jax.numpy as jnp
from jax import lax
from jax.experimental import pallas as pl
from jax.experimental.pallas import tpu as pltpu
from jax.experimental.pallas import tpu_sc as plsc

N_USERS = 25000
N_ITEMS = 25000
N = N_USERS + N_ITEMS
HALF = 32
E = 800000
BATCH = 4096
DECAY = 1e-4

NC = 2
NS = 16
NP = 50048
NNP = 2 * NP
ROWS_TILE = NP // NS
EP_TILE = E // NS
CHUNK = 400
SUB = 80
NSUB = CHUNK // SUB
NCHUNKS = EP_TILE // CHUNK
BP_TILE = BATCH // NS


def _sc_body(x0, pk, un, pn, nn, zrows,
             x1, x2, x3, diff_o, reg_o,
             acc, stg0, stg1, dv0, dv1, rows0, rows1,
             idx0_f, idxb_f, nd_f, sco_f, reg_v,
             stage_sem, gath_sem, scat0_sem, scat1_sem):
    c = lax.axis_index("c")
    s = lax.axis_index("s")
    iota = lax.iota(jnp.int32, 16)
    stg = (stg0, stg1)
    dvb = (dv0, dv1)
    rows = (rows0, rows1)
    scat_sem = (scat0_sem, scat1_sem)
    LAST = NCHUNKS - 1

    def spmm(in_ref, out_ref, interleaved):
        pltpu.sync_copy(zrows, acc.at[pl.ds(s * ROWS_TILE, ROWS_TILE)])
        plsc.subcore_barrier()

        def fire_stage(jj, b):
            off = (s * NCHUNKS + jj) * (3 * CHUNK)
            pltpu.async_copy(pk.at[pl.ds(off, 3 * CHUNK)], stg[b], stage_sem)

        def wait_stage(b):
            pltpu.make_async_copy(pk.at[pl.ds(0, 3 * CHUNK)], stg[b],
                                  stage_sem).wait()

        def adjust(b):
            sb, db = stg[b], dvb[b]
            for q in range(CHUNK // 16):
                sl = pl.ds(q * 16, 16)
                v = sb[sl]
                if interleaved:
                    sb[sl] = v * 2 + c
                else:
                    sb[sl] = v + c * NP
                db[q // (SUB // 16), pl.ds((q % (SUB // 16)) * 16, 16)] = (
                    sb[pl.ds(CHUNK + q * 16, 16)])

        def fire_gath(b):
            for k in range(NSUB):
                pltpu.async_copy(in_ref.at[stg[b].at[pl.ds(k * SUB, SUB)]],
                                 rows[b].at[pl.ds(k * SUB, SUB)], gath_sem)

        def wait_gath(b):
            for k in range(NSUB):
                pltpu.make_async_copy(
                    in_ref.at[stg[b].at[pl.ds(k * SUB, SUB)]],
                    rows[b].at[pl.ds(k * SUB, SUB)], gath_sem).wait()

        def fire_scat(b):
            for k in range(NSUB):
                pltpu.async_copy(rows[b].at[pl.ds(k * SUB, SUB)],
                                 acc.at[dvb[b].at[k]], scat_sem[b], add=True)

        def wait_scat(b):
            for k in range(NSUB):
                pltpu.make_async_copy(rows[b].at[pl.ds(k * SUB, SUB)],
                                      acc.at[dvb[b].at[k]],
                                      scat_sem[b]).wait()

        def compute(b):
            sb, rb = stg[b], rows[b]

            def grp(g, _):
                vv = plsc.bitcast(
                    plsc.load_gather(sb, [iota + 2 * CHUNK + g * 16]),
                    jnp.float32)
                eb = g * 16
                for e in range(16):
                    ve = vv.at[jnp.full((16,), e, jnp.int32)].get(
                        mode="promise_in_bounds")
                    r0 = rb[eb + e, pl.ds(0, 16)]
                    r1 = rb[eb + e, pl.ds(16, 16)]
                    rb[eb + e, pl.ds(0, 16)] = r0 * ve
                    rb[eb + e, pl.ds(16, 16)] = r1 * ve
                return 0

            lax.fori_loop(0, CHUNK // 16, grp, 0)

        def body(jj, b, first, last):
            nb = 1 - b
            if not last:
                fire_stage(jj + 1, nb)
            wait_gath(b)
            compute(b)
            if not last:
                wait_stage(nb)
                if not first:
                    wait_scat(nb)
                adjust(nb)
                fire_gath(nb)
            fire_scat(b)

        fire_stage(0, 0)
        wait_stage(0)
        adjust(0)
        fire_gath(0)
        body(jnp.int32(0), 0, True, False)
        body(jnp.int32(1), 1, False, False)

        def looped(t, _):
            body(2 * t + 2, 0, False, False)
            body(2 * t + 3, 1, False, False)
            return 0

        lax.fori_loop(0, (NCHUNKS - 3) // 2, looped, 0)
        body(jnp.int32(LAST), 0, False, True)
        wait_scat(1)
        wait_scat(0)

        plsc.subcore_barrier()
        pltpu.sync_copy(acc.at[pl.ds(s * ROWS_TILE, ROWS_TILE)],
                        out_ref.at[pl.ds(c * NP + s * ROWS_TILE, ROWS_TILE)])
        plsc.subcore_barrier()

    spmm(x0, x1, True)
    spmm(x1, x2, False)
    spmm(x2, x3, False)

    BP = 128
    reg = jnp.zeros((16,), jnp.float32)
    for p in range(2):
        for si, nodes_hbm in enumerate((un, pn, nn)):
            base = si * BP
            pltpu.sync_copy(nodes_hbm.at[pl.ds(s * BP_TILE + p * BP, BP)], nd_f)
            for q in range(BP // 16):
                sl = pl.ds(q * 16, 16)
                v = nd_f[sl]
                idx0_f[sl] = v * 2 + c
                idxb_f[sl] = v + c * NP
            pltpu.async_copy(x0.at[idx0_f], rows0.at[pl.ds(base, BP)],
                             gath_sem).wait()

            def regbody(i, r, base=base):
                a = rows0[base + i, pl.ds(0, 16)]
                b = rows0[base + i, pl.ds(16, 16)]
                return r + a * a + b * b

            reg = lax.fori_loop(0, BP, regbody, reg)

            for tab in (x1, x2, x3):
                pltpu.async_copy(tab.at[idxb_f], rows1.at[pl.ds(0, BP)],
                                 gath_sem).wait()

                def addbody(i, _, base=base):
                    rows0[base + i, pl.ds(0, 16)] = (
                        rows0[base + i, pl.ds(0, 16)] + rows1[i, pl.ds(0, 16)])
                    rows0[base + i, pl.ds(16, 16)] = (
                        rows0[base + i, pl.ds(16, 16)] + rows1[i, pl.ds(16, 16)])
                    return 0

                lax.fori_loop(0, BP, addbody, 0)

        def score(g, _, p=p):
            eids = iota + g * 16
            ps = jnp.zeros((16,), jnp.float32)
            ns = jnp.zeros((16,), jnp.float32)
            for d in range(HALF):
                dv = jnp.full((16,), d, jnp.int32)
                cu = plsc.load_gather(rows0, [eids, dv])
                cp = plsc.load_gather(rows0, [eids + BP, dv])
                cn = plsc.load_gather(rows0, [eids + 2 * BP, dv])
                ps = ps + cu * cp
                ns = ns + cu * cn
            plsc.store_scatter(sco_f, [eids + p * BP], (ps - ns) * 0.0625)
            return 0

        lax.fori_loop(0, BP // 16, score, 0)

    reg_v[...] = reg
    pltpu.sync_copy(reg_v, reg_o.at[pl.ds((c * NS + s) * 16, 16)])
    pltpu.sync_copy(sco_f, diff_o.at[pl.ds((c * NS + s) * BP_TILE, BP_TILE)])


def _tc_final(diff_ref, reg_ref, loss_ref, bpr_ref):
    sd = diff_ref[0, :] + diff_ref[1, :]
    bpr = -jnp.mean(jax.nn.log_sigmoid(sd))
    reg = jnp.sum(reg_ref[...]) / BATCH
    loss_ref[0, 0] = bpr + DECAY * reg
    bpr_ref[0, 0] = bpr


@jax.jit
def kernel(user_emb_w, item_emb_w, adj_values, users, pos_items, neg_items, adj_indices):
    x0 = jnp.concatenate([user_emb_w, item_emb_w], axis=0).reshape(2 * N, HALF)
    src = adj_indices[0].astype(jnp.int32).reshape(NS, NCHUNKS, CHUNK)
    dst = adj_indices[1].astype(jnp.int32).reshape(NS, NCHUNKS, CHUNK)
    vali = jax.lax.bitcast_convert_type(adj_values, jnp.int32).reshape(
        NS, NCHUNKS, CHUNK)
    pk = jnp.stack((src, dst, vali), axis=2).reshape(-1)
    un = users.astype(jnp.int32)
    pn = pos_items.astype(jnp.int32) + N_USERS
    nn = neg_items.astype(jnp.int32) + N_USERS
    zrows = jnp.zeros((ROWS_TILE, HALF), jnp.float32)

    mesh = plsc.VectorSubcoreMesh(core_axis_name="c", subcore_axis_name="s",
                                  num_cores=NC, num_subcores=NS)
    f32 = jnp.float32
    i32 = jnp.int32
    sc = pl.kernel(
        _sc_body,
        out_type=(
            jax.ShapeDtypeStruct((NNP, HALF), f32),
            jax.ShapeDtypeStruct((NNP, HALF), f32),
            jax.ShapeDtypeStruct((NNP, HALF), f32),
            jax.ShapeDtypeStruct((NC * BATCH,), f32),
            jax.ShapeDtypeStruct((NC * NS * 16,), f32),
        ),
        mesh=mesh,
        compiler_params=pltpu.CompilerParams(needs_layout_passes=False,
                                             use_tc_tiling_on_sc=False),
        scratch_types=[
            pltpu.VMEM_SHARED((NP, HALF), f32),
            pltpu.VMEM((3 * CHUNK,), i32),
            pltpu.VMEM((3 * CHUNK,), i32),
            pltpu.VMEM((NSUB, SUB), i32),
            pltpu.VMEM((NSUB, SUB), i32),
            pltpu.VMEM((CHUNK, HALF), f32),
            pltpu.VMEM((CHUNK, HALF), f32),
            pltpu.VMEM((128,), i32),
            pltpu.VMEM((128,), i32),
            pltpu.VMEM((128,), i32),
            pltpu.VMEM((BP_TILE,), f32),
            pltpu.VMEM((16,), f32),
            pltpu.SemaphoreType.DMA,
            pltpu.SemaphoreType.DMA,
            pltpu.SemaphoreType.DMA,
            pltpu.SemaphoreType.DMA,
        ],
    )
    x1, x2, x3, diff_p, reg_p = sc(x0, pk, un, pn, nn, zrows)

    diff = diff_p.reshape(NC, BATCH)
    regs = reg_p.reshape(4, 128)
    loss, bpr = pl.pallas_call(
        _tc_final,
        out_shape=(
            jax.ShapeDtypeStruct((1, 1), f32),
            jax.ShapeDtypeStruct((1, 1), f32),
        ),
        out_specs=(pl.BlockSpec(memory_space=pltpu.SMEM),
                   pl.BlockSpec(memory_space=pltpu.SMEM)),
    )(diff, regs)
    return (loss[0, 0], bpr[0, 0])


# speedup vs baseline: 13.1536x; 1.1802x over previous
import jax
import jax.numpy as jnp
from jax import lax
from jax.experimental import pallas as pl
from jax.experimental.pallas import tpu as pltpu
from jax.experimental.pallas import tpu_sc as plsc

N_USERS = 25000
N_ITEMS = 25000
N = N_USERS + N_ITEMS
HALF = 32
E = 800000
BATCH = 4096
DECAY = 1e-4

NC = 2
NS = 16
NP = 50048
NNP = 2 * NP
ROWS_TILE = NP // NS
EP_TILE = E // NS
CHUNK = 400
SUB = 80
NSUB = CHUNK // SUB
NCHUNKS = EP_TILE // CHUNK
BP_TILE = BATCH // NS
BP = 128


def _sc_body(x0, pk, un, pn, nn, zrows,
             x1, x2, x3, um_o, pm_o, nm_o, reg_o,
             acc, stg0, stg1, stg2, dv0, dv1, rows0, rows1, reg_v,
             stage_sem, gath_sem, scat0_sem, scat1_sem):
    c = lax.axis_index("c")
    s = lax.axis_index("s")
    iota = lax.iota(jnp.int32, 16)
    stg = (stg0, stg1, stg2)
    dvb = (dv0, dv1)
    rows = (rows0, rows1)
    scat_sem = (scat0_sem, scat1_sem)
    LAST = NCHUNKS - 1

    def spmm(in_ref, out_ref, interleaved):
        pltpu.sync_copy(zrows, acc.at[pl.ds(s * ROWS_TILE, ROWS_TILE)])
        plsc.subcore_barrier()

        def fire_stage(jj, si):
            off = (s * NCHUNKS + jj) * (3 * CHUNK)
            pltpu.async_copy(pk.at[pl.ds(off, 3 * CHUNK)], stg[si], stage_sem)

        def wait_stage(si):
            pltpu.make_async_copy(pk.at[pl.ds(0, 3 * CHUNK)], stg[si],
                                  stage_sem).wait()

        def adjust(si, dp):
            sb, db = stg[si], dvb[dp]

            def ab(q, _):
                idx = iota + q * 16
                v = plsc.load_gather(sb, [idx])
                if interleaved:
                    v = v * 2 + c
                else:
                    v = v + c * NP
                plsc.store_scatter(sb, [idx], v)
                d = plsc.load_gather(sb, [idx + CHUNK])
                plsc.store_scatter(
                    db, [jnp.full((16,), 0, jnp.int32) + q // (SUB // 16),
                         iota + (q % (SUB // 16)) * 16], d)
                return 0

            lax.fori_loop(0, CHUNK // 16, ab, 0)

        def fire_gath(si, rp):
            for k in range(NSUB):
                pltpu.async_copy(in_ref.at[stg[si].at[pl.ds(k * SUB, SUB)]],
                                 rows[rp].at[pl.ds(k * SUB, SUB)], gath_sem)

        def wait_gath(si, rp):
            for k in range(NSUB):
                pltpu.make_async_copy(
                    in_ref.at[stg[si].at[pl.ds(k * SUB, SUB)]],
                    rows[rp].at[pl.ds(k * SUB, SUB)], gath_sem).wait()

        def fire_scat(rp):
            for k in range(NSUB):
                pltpu.async_copy(rows[rp].at[pl.ds(k * SUB, SUB)],
                                 acc.at[dvb[rp].at[k]], scat_sem[rp],
                                 add=True)

        def wait_scat(rp):
            for k in range(NSUB):
                pltpu.make_async_copy(rows[rp].at[pl.ds(k * SUB, SUB)],
                                      acc.at[dvb[rp].at[k]],
                                      scat_sem[rp]).wait()

        def compute(rp, si):
            sb, rb = stg[si], rows[rp]

            def grp(g, _):
                vv = plsc.bitcast(
                    plsc.load_gather(sb, [iota + 2 * CHUNK + g * 16]),
                    jnp.float32)
                eb = g * 16
                for e in range(16):
                    ve = vv.at[jnp.full((16,), e, jnp.int32)].get(
                        mode="promise_in_bounds")
                    r0 = rb[eb + e, pl.ds(0, 16)]
                    r1 = rb[eb + e, pl.ds(16, 16)]
                    rb[eb + e, pl.ds(0, 16)] = r0 * ve
                    rb[eb + e, pl.ds(16, 16)] = r1 * ve
                return 0

            lax.fori_loop(0, CHUNK // 16, grp, 0)

        def body(jj, ip, first=False, last=False):
            rbp = ip % 2
            s0, s1, s2 = ip % 3, (ip + 1) % 3, (ip + 2) % 3
            wait_gath(s0, rbp)
            if not first:
                wait_scat(1 - rbp)
            if not last:
                wait_stage(s1)
                adjust(s1, 1 - rbp)
                fire_gath(s1, 1 - rbp)

                @pl.when(jj + 2 <= LAST)
                def _():
                    fire_stage(jj + 2, s2)
            compute(rbp, s0)
            fire_scat(rbp)

        fire_stage(jnp.int32(0), 0)
        fire_stage(jnp.int32(1), 1)
        wait_stage(0)
        adjust(0, 0)
        fire_gath(0, 0)
        body(jnp.int32(0), 0, first=True)
        body(jnp.int32(1), 1)

        def looped(t, _):
            jj = 2 + 6 * t
            for i in range(6):
                body(jj + i, 2 + i)
            return 0

        lax.fori_loop(0, (NCHUNKS - 5) // 6, looped, 0)
        body(jnp.int32(NCHUNKS - 3), NCHUNKS - 3)
        body(jnp.int32(NCHUNKS - 2), NCHUNKS - 2)
        body(jnp.int32(LAST), LAST, last=True)
        wait_scat(LAST % 2)

        plsc.subcore_barrier()
        pltpu.sync_copy(acc.at[pl.ds(s * ROWS_TILE, ROWS_TILE)],
                        out_ref.at[pl.ds(c * NP + s * ROWS_TILE, ROWS_TILE)])
        plsc.subcore_barrier()

    spmm(x0, x1, True)
    spmm(x1, x2, False)
    spmm(x2, x3, False)

    reg = jnp.zeros((16,), jnp.float32)
    for p in range(2):
        outcps = []
        for si_set, (nodes_hbm, m_out) in enumerate(
                ((un, um_o), (pn, pm_o), (nn, nm_o))):
            base = si_set * BP
            pltpu.sync_copy(nodes_hbm.at[pl.ds(s * BP_TILE + p * BP, BP)],
                            stg0.at[pl.ds(0, BP)])
            for q in range(BP // 16):
                sl = pl.ds(q * 16, 16)
                v = stg0[sl]
                stg0[pl.ds(BP + q * 16, 16)] = v * 2 + c
                stg0[pl.ds(2 * BP + q * 16, 16)] = v + c * NP
            hs = [pltpu.async_copy(x0.at[stg0.at[pl.ds(BP, BP)]],
                                   rows0.at[pl.ds(base, BP)], gath_sem)]
            for ti, tab in enumerate((x1, x2, x3)):
                hs.append(pltpu.async_copy(
                    tab.at[stg0.at[pl.ds(2 * BP, BP)]],
                    rows1.at[pl.ds(ti * BP, BP)], gath_sem))
            for h in hs:
                h.wait()

            def addbody(i, r, base=base):
                m0 = rows0[base + i, pl.ds(0, 16)]
                m1 = rows0[base + i, pl.ds(16, 16)]
                r = r + m0 * m0 + m1 * m1
                for ti in range(3):
                    m0 = m0 + rows1[ti * BP + i, pl.ds(0, 16)]
                    m1 = m1 + rows1[ti * BP + i, pl.ds(16, 16)]
                rows0[base + i, pl.ds(0, 16)] = m0
                rows0[base + i, pl.ds(16, 16)] = m1
                return r

            reg = lax.fori_loop(0, BP, addbody, reg)
            outcps.append(pltpu.async_copy(
                rows0.at[pl.ds(base, BP)],
                m_out.at[pl.ds(c * BATCH + s * BP_TILE + p * BP, BP)],
                stage_sem))
        for h in outcps:
            h.wait()

    reg_v[...] = reg
    pltpu.sync_copy(reg_v, reg_o.at[pl.ds((c * NS + s) * 16, 16)])


def _tc_final(u_ref, p_ref, n_ref, reg_ref, loss_ref, bpr_ref):
    u = u_ref[...]
    sd = jnp.sum(u * (p_ref[...] - n_ref[...]), axis=-1)
    sd = (sd[0] + sd[1]) * 0.0625
    bpr = -jnp.mean(jax.nn.log_sigmoid(sd))
    reg = jnp.sum(reg_ref[...]) / BATCH
    loss_ref[0, 0] = bpr + DECAY * reg
    bpr_ref[0, 0] = bpr


@jax.jit
def kernel(user_emb_w, item_emb_w, adj_values, users, pos_items, neg_items, adj_indices):
    x0 = jnp.concatenate([user_emb_w, item_emb_w], axis=0).reshape(2 * N, HALF)
    src = adj_indices[0].astype(jnp.int32).reshape(NS, NCHUNKS, CHUNK)
    dst = adj_indices[1].astype(jnp.int32).reshape(NS, NCHUNKS, CHUNK)
    vali = jax.lax.bitcast_convert_type(adj_values, jnp.int32).reshape(
        NS, NCHUNKS, CHUNK)
    pk = jnp.stack((src, dst, vali), axis=2).reshape(-1)
    un = users.astype(jnp.int32)
    pn = pos_items.astype(jnp.int32) + N_USERS
    nn = neg_items.astype(jnp.int32) + N_USERS
    zrows = jnp.zeros((ROWS_TILE, HALF), jnp.float32)

    mesh = plsc.VectorSubcoreMesh(core_axis_name="c", subcore_axis_name="s",
                                  num_cores=NC, num_subcores=NS)
    f32 = jnp.float32
    i32 = jnp.int32
    sc = pl.kernel(
        _sc_body,
        out_type=(
            jax.ShapeDtypeStruct((NNP, HALF), f32),
            jax.ShapeDtypeStruct((NNP, HALF), f32),
            jax.ShapeDtypeStruct((NNP, HALF), f32),
            jax.ShapeDtypeStruct((NC * BATCH, HALF), f32),
            jax.ShapeDtypeStruct((NC * BATCH, HALF), f32),
            jax.ShapeDtypeStruct((NC * BATCH, HALF), f32),
            jax.ShapeDtypeStruct((NC * NS * 16,), f32),
        ),
        mesh=mesh,
        compiler_params=pltpu.CompilerParams(needs_layout_passes=False,
                                             use_tc_tiling_on_sc=False),
        scratch_types=[
            pltpu.VMEM_SHARED((NP, HALF), f32),
            pltpu.VMEM((3 * CHUNK,), i32),
            pltpu.VMEM((3 * CHUNK,), i32),
            pltpu.VMEM((3 * CHUNK,), i32),
            pltpu.VMEM((NSUB, SUB), i32),
            pltpu.VMEM((NSUB, SUB), i32),
            pltpu.VMEM((CHUNK, HALF), f32),
            pltpu.VMEM((CHUNK, HALF), f32),
            pltpu.VMEM((16,), f32),
            pltpu.SemaphoreType.DMA,
            pltpu.SemaphoreType.DMA,
            pltpu.SemaphoreType.DMA,
            pltpu.SemaphoreType.DMA,
        ],
    )
    x1, x2, x3, um, pm, nm, reg_p = sc(x0, pk, un, pn, nn, zrows)

    loss, bpr = pl.pallas_call(
        _tc_final,
        out_shape=(
            jax.ShapeDtypeStruct((1, 1), f32),
            jax.ShapeDtypeStruct((1, 1), f32),
        ),
        out_specs=(pl.BlockSpec(memory_space=pltpu.SMEM),
                   pl.BlockSpec(memory_space=pltpu.SMEM)),
    )(um.reshape(NC, BATCH, HALF), pm.reshape(NC, BATCH, HALF),
      nm.reshape(NC, BATCH, HALF), reg_p.reshape(4, 128))
    return (loss[0, 0], bpr[0, 0])
